# initial kernel scaffold (unmeasured)
import jax
import jax.numpy as jnp
from jax import lax
from jax.experimental import pallas as pl
from jax.experimental.pallas import tpu as pltpu


def kernel(ids, E):
    v_per, d = E.shape
    t = ids.shape[0]

    my_z = lax.axis_index("z")
    off = my_z * v_per
    local = ids - off
    mask = (local >= 0) & (local < v_per)
    safe = jnp.where(mask, local, 0)
    partial = jnp.where(mask[:, None], E[safe], 0.0).astype(jnp.bfloat16)

    def body(p_ref, out_ref, comm_ref, send_sem, recv_sem):
        x = lax.axis_index("x")
        y = lax.axis_index("y")
        z = lax.axis_index("z")
        partner = (x, y, 1 - z)

        barrier_sem = pltpu.get_barrier_semaphore()
        pl.semaphore_signal(
            barrier_sem, inc=1,
            device_id=partner, device_id_type=pl.DeviceIdType.MESH,
        )
        pl.semaphore_wait(barrier_sem, 1)

        rdma = pltpu.make_async_remote_copy(
            src_ref=p_ref,
            dst_ref=comm_ref,
            send_sem=send_sem,
            recv_sem=recv_sem,
            device_id=partner,
            device_id_type=pl.DeviceIdType.MESH,
        )
        rdma.start()
        rdma.wait()

        out_ref[...] = p_ref[...].astype(jnp.float32) + comm_ref[...].astype(
            jnp.float32
        )

    return pl.pallas_call(
        body,
        out_shape=jax.ShapeDtypeStruct((t, d), jnp.float32),
        in_specs=[pl.BlockSpec(memory_space=pltpu.VMEM)],
        out_specs=pl.BlockSpec(memory_space=pltpu.VMEM),
        scratch_shapes=[
            pltpu.VMEM((t, d), jnp.bfloat16),
            pltpu.SemaphoreType.DMA,
            pltpu.SemaphoreType.DMA,
        ],
        compiler_params=pltpu.CompilerParams(collective_id=0),
    )(partial)


# baseline (device time: 52862 ns/iter reference)
import jax
import jax.numpy as jnp
from jax import lax
from jax.experimental import pallas as pl
from jax.experimental.pallas import tpu as pltpu

UNROLL = 8


def kernel(ids, E):
    v_per, d = E.shape
    t = ids.shape[0]

    ids_2d = ids.reshape(t, 1)

    def body(ids_sm, ids_v, e_hbm, out_ref, gath, partial, comm,
             gsem, send_sem, recv_sem):
        x = lax.axis_index("x")
        y = lax.axis_index("y")
        z = lax.axis_index("z")
        partner = (x, y, 1 - z)
        off = z * v_per

        barrier_sem = pltpu.get_barrier_semaphore()
        pl.semaphore_signal(
            barrier_sem, inc=1,
            device_id=partner, device_id_type=pl.DeviceIdType.MESH,
        )

        def issue(i, _):
            base = i * UNROLL
            for u in range(UNROLL):
                tok = base + u
                safe = jnp.clip(ids_sm[tok] - off, 0, v_per - 1)
                pltpu.make_async_copy(
                    e_hbm.at[pl.ds(safe, 1), :],
                    gath.at[pl.ds(tok, 1), :],
                    gsem,
                ).start()
            return 0

        lax.fori_loop(0, t // UNROLL, issue, 0, unroll=False)

        pltpu.make_async_copy(e_hbm.at[pl.ds(0, t), :], gath, gsem).wait()

        mask = (ids_v[...] >= off) & (ids_v[...] < off + v_per)
        partial[...] = jnp.where(mask, gath[...], 0.0).astype(jnp.bfloat16)

        pl.semaphore_wait(barrier_sem, 1)

        rdma = pltpu.make_async_remote_copy(
            src_ref=partial,
            dst_ref=comm,
            send_sem=send_sem,
            recv_sem=recv_sem,
            device_id=partner,
            device_id_type=pl.DeviceIdType.MESH,
        )
        rdma.start()
        rdma.wait()

        out_ref[...] = partial[...].astype(jnp.float32) + comm[...].astype(
            jnp.float32
        )

    return pl.pallas_call(
        body,
        out_shape=jax.ShapeDtypeStruct((t, d), jnp.float32),
        in_specs=[
            pl.BlockSpec(memory_space=pltpu.SMEM),
            pl.BlockSpec(memory_space=pltpu.VMEM),
            pl.BlockSpec(memory_space=pltpu.MemorySpace.HBM),
        ],
        out_specs=pl.BlockSpec(memory_space=pltpu.VMEM),
        scratch_shapes=[
            pltpu.VMEM((t, d), jnp.float32),
            pltpu.VMEM((t, d), jnp.bfloat16),
            pltpu.VMEM((t, d), jnp.bfloat16),
            pltpu.SemaphoreType.DMA,
            pltpu.SemaphoreType.DMA,
            pltpu.SemaphoreType.DMA,
        ],
        compiler_params=pltpu.CompilerParams(collective_id=0),
    )(ids, ids_2d, E)


# device time: 34274 ns/iter; 1.5423x vs baseline; 1.5423x over previous
import jax
import jax.numpy as jnp
from jax import lax
from jax.experimental import pallas as pl
from jax.experimental.pallas import tpu as pltpu

C = 8
UNROLL = 8


def kernel(ids, E):
    v_per, d = E.shape
    t = ids.shape[0]
    ch = t // C

    my_z = lax.axis_index("z")
    off = my_z * v_per
    safe = jnp.clip(ids - off, 0, v_per - 1)
    ids_2d = ids.reshape(t, 1)

    def body(safe_sm, ids_v, e_hbm, out_ref, gath, partial, comm,
             gsems, send_sems, recv_sems):
        x = lax.axis_index("x")
        y = lax.axis_index("y")
        z = lax.axis_index("z")
        partner = (x, y, 1 - z)
        voff = z * v_per

        barrier_sem = pltpu.get_barrier_semaphore()
        pl.semaphore_signal(
            barrier_sem, inc=1,
            device_id=partner, device_id_type=pl.DeviceIdType.MESH,
        )

        def issue_chunk(k):
            def issue(i, _):
                base = k * ch + i * UNROLL
                for u in range(UNROLL):
                    tok = base + u
                    pltpu.make_async_copy(
                        e_hbm.at[pl.ds(safe_sm[tok], 1), :],
                        gath.at[pl.ds(tok, 1), :],
                        gsems.at[k],
                    ).start()
                return 0
            lax.fori_loop(0, ch // UNROLL, issue, 0, unroll=False)

        def chunk_rdma(k):
            return pltpu.make_async_remote_copy(
                src_ref=partial.at[pl.ds(k * ch, ch), :],
                dst_ref=comm.at[pl.ds(k * ch, ch), :],
                send_sem=send_sems.at[k],
                recv_sem=recv_sems.at[k],
                device_id=partner,
                device_id_type=pl.DeviceIdType.MESH,
            )

        issue_chunk(0)
        pl.semaphore_wait(barrier_sem, 1)

        for k in range(C):
            if k + 1 < C:
                issue_chunk(k + 1)
            pltpu.make_async_copy(
                e_hbm.at[pl.ds(0, ch), :],
                gath.at[pl.ds(k * ch, ch), :],
                gsems.at[k],
            ).wait()
            sl = pl.ds(k * ch, ch)
            mask = (ids_v[sl] >= voff) & (ids_v[sl] < voff + v_per)
            partial[sl, :] = jnp.where(mask, gath[sl, :], 0.0).astype(
                jnp.bfloat16
            )
            chunk_rdma(k).start()

        for k in range(C):
            chunk_rdma(k).wait()
            sl = pl.ds(k * ch, ch)
            out_ref[sl, :] = partial[sl, :].astype(jnp.float32) + comm[
                sl, :
            ].astype(jnp.float32)

    return pl.pallas_call(
        body,
        out_shape=jax.ShapeDtypeStruct((t, d), jnp.float32),
        in_specs=[
            pl.BlockSpec(memory_space=pltpu.SMEM),
            pl.BlockSpec(memory_space=pltpu.VMEM),
            pl.BlockSpec(memory_space=pltpu.MemorySpace.HBM),
        ],
        out_specs=pl.BlockSpec(memory_space=pltpu.VMEM),
        scratch_shapes=[
            pltpu.VMEM((t, d), jnp.float32),
            pltpu.VMEM((t, d), jnp.bfloat16),
            pltpu.VMEM((t, d), jnp.bfloat16),
            pltpu.SemaphoreType.DMA((C,)),
            pltpu.SemaphoreType.DMA((C,)),
            pltpu.SemaphoreType.DMA((C,)),
        ],
        compiler_params=pltpu.CompilerParams(collective_id=0),
    )(safe, ids_2d, E)
